# Initial kernel scaffold; baseline (speedup 1.0000x reference)
import jax
import jax.numpy as jnp
from jax.experimental import pallas as pl


def kernel(token_ids, emb_s1, emb_s2, W_fuse, b_fuse):
    raise NotImplementedError("write your pallas kernel here")


# speedup vs baseline: 3.0403x; 3.0403x over previous
import functools
import math

import jax
import jax.numpy as jnp
from jax import lax
from jax.experimental import pallas as pl
from jax.experimental.pallas import tpu as pltpu
from jax.experimental.pallas import tpu_sc as plsc

D = 256
VOCAB = 8192
SCALE = math.sqrt(D)
S2_BITS = 13
S2_MASK = (1 << S2_BITS) - 1


_ROWS_PER_BLOCK = 1024
_N_BLOCKS = VOCAB // _ROWS_PER_BLOCK


def _transform_body(e1_ref, e2_ref, w1_ref, w2_ref, b_ref, t1_ref, t2_ref):
    t1_ref[...] = (
        jnp.dot(e1_ref[...], w1_ref[...], preferred_element_type=jnp.float32)
        * SCALE
        + b_ref[...]
    )
    t2_ref[...] = (
        jnp.dot(e2_ref[...], w2_ref[...], preferred_element_type=jnp.float32)
        * SCALE
    )


def _transform_tables(emb_s1, emb_s2, w1, w2, b2d):
    blk = _ROWS_PER_BLOCK
    return pl.pallas_call(
        _transform_body,
        grid=(_N_BLOCKS,),
        in_specs=[
            pl.BlockSpec((blk, D), lambda i: (i, 0)),
            pl.BlockSpec((blk, D), lambda i: (i, 0)),
            pl.BlockSpec((D, D), lambda i: (0, 0)),
            pl.BlockSpec((D, D), lambda i: (0, 0)),
            pl.BlockSpec((1, D), lambda i: (0, 0)),
        ],
        out_specs=[
            pl.BlockSpec((blk, D), lambda i: (i, 0)),
            pl.BlockSpec((blk, D), lambda i: (i, 0)),
        ],
        out_shape=[
            jax.ShapeDtypeStruct((VOCAB, D), jnp.float32),
            jax.ShapeDtypeStruct((VOCAB, D), jnp.float32),
        ],
    )(emb_s1, emb_s2, w1, w2, b2d)


_NTOK = 32768
_NW = 32
_TPW = _NTOK // _NW
_C = 128
_NCHUNK = _TPW // _C
_L = 16


def _gather_add_body(tok_hbm, t1_hbm, t2_hbm, out_hbm,
                     tok_v, idx1_v, idx2_v, r1_v, r2_v, sem):
    wid = lax.axis_index("s") * 2 + lax.axis_index("c")
    base = wid * _TPW

    pltpu.sync_copy(tok_hbm.at[pl.ds(base, _TPW)], tok_v)

    def split_body(j, _):
        sl = pl.ds(j * _L, _L)
        t = tok_v[sl]
        idx1_v[sl] = lax.shift_right_logical(t, S2_BITS)
        idx2_v[sl] = lax.bitwise_and(t, S2_MASK)
        return 0

    lax.fori_loop(0, _TPW // _L, split_body, 0)

    for c in range(_NCHUNK):
        i1 = idx1_v.at[pl.ds(c * _C, _C)]
        i2 = idx2_v.at[pl.ds(c * _C, _C)]
        g1 = pltpu.async_copy(t1_hbm.at[i1], r1_v, sem)
        g1.wait()
        g2 = pltpu.async_copy(t2_hbm.at[i2], r2_v, sem)
        g2.wait()

        def add_row(r, _):
            for k in range(D // _L):
                sl = pl.ds(k * _L, _L)
                plsc.addupdate(r1_v.at[r, sl], r2_v[r, sl])
            return 0

        lax.fori_loop(0, _C, add_row, 0)

        pltpu.sync_copy(r1_v, out_hbm.at[pl.ds(base + c * _C, _C)])


def _gather_add(tok, t1, t2):
    mesh = plsc.VectorSubcoreMesh(core_axis_name="c", subcore_axis_name="s")
    fn = functools.partial(
        pl.kernel,
        mesh=mesh,
        out_type=jax.ShapeDtypeStruct((_NTOK, D), jnp.float32),
        scratch_types=[
            pltpu.VMEM((_TPW,), jnp.int32),
            pltpu.VMEM((_TPW,), jnp.int32),
            pltpu.VMEM((_TPW,), jnp.int32),
            pltpu.VMEM((_C, D), jnp.float32),
            pltpu.VMEM((_C, D), jnp.float32),
            pltpu.SemaphoreType.DMA,
        ],
    )(_gather_add_body)
    return fn(tok, t1, t2)


def kernel(token_ids, emb_s1, emb_s2, W_fuse, b_fuse):
    w1 = W_fuse[:D]
    w2 = W_fuse[D:]
    b2d = b_fuse.reshape(1, D)
    t1, t2 = _transform_tables(emb_s1, emb_s2, w1, w2, b2d)
    tok = token_ids.reshape(-1)
    out = _gather_add(tok, t1, t2)
    return out.reshape(token_ids.shape + (D,))


# speedup vs baseline: 3.9893x; 1.3121x over previous
import functools
import math

import jax
import jax.numpy as jnp
from jax import lax
from jax.experimental import pallas as pl
from jax.experimental.pallas import tpu as pltpu
from jax.experimental.pallas import tpu_sc as plsc

D = 256
VOCAB = 8192
SCALE = math.sqrt(D)
S2_BITS = 13
S2_MASK = (1 << S2_BITS) - 1


_ROWS_PER_BLOCK = 1024
_N_BLOCKS = VOCAB // _ROWS_PER_BLOCK


def _transform_body(e1_ref, e2_ref, w1_ref, w2_ref, b_ref, t1_ref, t2_ref):
    t1_ref[...] = (
        jnp.dot(e1_ref[...], w1_ref[...], preferred_element_type=jnp.float32)
        * SCALE
        + b_ref[...]
    )
    t2_ref[...] = (
        jnp.dot(e2_ref[...], w2_ref[...], preferred_element_type=jnp.float32)
        * SCALE
    )


def _transform_tables(emb_s1, emb_s2, w1, w2, b2d):
    blk = _ROWS_PER_BLOCK
    return pl.pallas_call(
        _transform_body,
        grid=(_N_BLOCKS,),
        in_specs=[
            pl.BlockSpec((blk, D), lambda i: (i, 0)),
            pl.BlockSpec((blk, D), lambda i: (i, 0)),
            pl.BlockSpec((D, D), lambda i: (0, 0)),
            pl.BlockSpec((D, D), lambda i: (0, 0)),
            pl.BlockSpec((1, D), lambda i: (0, 0)),
        ],
        out_specs=[
            pl.BlockSpec((blk, D), lambda i: (i, 0)),
            pl.BlockSpec((blk, D), lambda i: (i, 0)),
        ],
        out_shape=[
            jax.ShapeDtypeStruct((VOCAB, D), jnp.float32),
            jax.ShapeDtypeStruct((VOCAB, D), jnp.float32),
        ],
    )(emb_s1, emb_s2, w1, w2, b2d)


_NTOK = 32768
_NW = 32
_TPW = _NTOK // _NW
_C = 64
_NCHUNK = _TPW // _C
_L = 16


def _gather_add_body(tok_hbm, t1_hbm, t2_hbm, out_hbm,
                     tok_v, idx1_v, idx2_v,
                     r1_v, r2_v, o_v, sem_g, sem_o):
    wid = lax.axis_index("s") * 2 + lax.axis_index("c")
    base = wid * _TPW

    pltpu.sync_copy(tok_hbm.at[pl.ds(base, _TPW)], tok_v)

    def split_body(j, _):
        sl = pl.ds(j * _L, _L)
        t = tok_v[sl]
        idx1_v[sl] = lax.shift_right_logical(t, S2_BITS)
        idx2_v[sl] = lax.bitwise_and(t, S2_MASK)
        return 0

    lax.fori_loop(0, _TPW // _L, split_body, 0)

    def fire_gathers(c, p):
        i1 = idx1_v.at[pl.ds(c * _C, _C)]
        i2 = idx2_v.at[pl.ds(c * _C, _C)]
        g1 = pltpu.async_copy(t1_hbm.at[i1], r1_v.at[p], sem_g[p])
        g2 = pltpu.async_copy(t2_hbm.at[i2], r2_v.at[p], sem_g[p])
        return g1, g2

    pending_g = [None, None]
    pending_o = [None, None]
    pending_g[0] = fire_gathers(0, 0)

    for c in range(_NCHUNK):
        p = c & 1
        q = 1 - p
        if c + 1 < _NCHUNK:
            pending_g[q] = fire_gathers(c + 1, q)
        g1, g2 = pending_g[p]
        g1.wait()
        g2.wait()
        if pending_o[p] is not None:
            pending_o[p].wait()

        def add_row(r, _):
            for k in range(D // _L):
                sl = pl.ds(k * _L, _L)
                o_v[p, r, sl] = r1_v[p, r, sl] + r2_v[p, r, sl]
            return 0

        lax.fori_loop(0, _C, add_row, 0)

        pending_o[p] = pltpu.async_copy(
            o_v.at[p], out_hbm.at[pl.ds(base + c * _C, _C)], sem_o[p])

    pending_o[0].wait()
    pending_o[1].wait()


def _gather_add(tok, t1, t2):
    mesh = plsc.VectorSubcoreMesh(core_axis_name="c", subcore_axis_name="s")
    fn = functools.partial(
        pl.kernel,
        mesh=mesh,
        out_type=jax.ShapeDtypeStruct((_NTOK, D), jnp.float32),
        scratch_types=[
            pltpu.VMEM((_TPW,), jnp.int32),
            pltpu.VMEM((_TPW,), jnp.int32),
            pltpu.VMEM((_TPW,), jnp.int32),
            pltpu.VMEM((2, _C, D), jnp.float32),
            pltpu.VMEM((2, _C, D), jnp.float32),
            pltpu.VMEM((2, _C, D), jnp.float32),
            [pltpu.SemaphoreType.DMA, pltpu.SemaphoreType.DMA],
            [pltpu.SemaphoreType.DMA, pltpu.SemaphoreType.DMA],
        ],
    )(_gather_add_body)
    return fn(tok, t1, t2)


def kernel(token_ids, emb_s1, emb_s2, W_fuse, b_fuse):
    w1 = W_fuse[:D]
    w2 = W_fuse[D:]
    b2d = b_fuse.reshape(1, D)
    t1, t2 = _transform_tables(emb_s1, emb_s2, w1, w2, b2d)
    tok = token_ids.reshape(-1)
    out = _gather_add(tok, t1, t2)
    return out.reshape(token_ids.shape + (D,))
